# baseline (speedup 1.0000x reference)
import functools

import jax
import jax.numpy as jnp
from jax import lax
from jax.experimental import pallas as pl
from jax.experimental.pallas import tpu as pltpu
from jax.experimental.pallas import tpu_sc as plsc

N_FIELDS = 26
VOCAB = 100000
EMBED = 32
N_NUM = 13
N_TOK = N_FIELDS + N_NUM
LANES = 16


@functools.partial(jax.jit, static_argnames=("batch",))
def _run(xcat_flat, xnum_flat, tbl, w, bvec, *, batch):
    info = plsc.get_sparse_core_info()
    nc, ns = info.num_cores, info.num_subcores
    nw = nc * ns
    rows_per_w = batch // nw
    chunk = 64
    n_chunks = rows_per_w // chunk

    mesh = plsc.VectorSubcoreMesh(core_axis_name="c", subcore_axis_name="s")

    def body(xcat_ref, xnum_ref, tbl_ref, w_ref, b_ref, out_ref,
             ridx_v, fidx_v, x_v, g_v, n_v, w_v, b_v, gsem, wsem):
        wid = lax.axis_index("s") * nc + lax.axis_index("c")

        pltpu.sync_copy(w_ref, w_v)
        pltpu.sync_copy(b_ref, b_v)
        wlo = w_v[pl.ds(0, LANES)]
        whi = w_v[pl.ds(LANES, LANES)]
        blo = b_v[pl.ds(0, LANES)]
        bhi = b_v[pl.ds(LANES, LANES)]
        iota26 = lax.iota(jnp.int32, LANES) * N_FIELDS

        def drain_writes(base, c):
            def f_wd(f, cc):
                pltpu.make_async_copy(
                    g_v.at[f], out_ref.at[pl.ds(base, chunk), f], wsem).wait()
                return cc

            lax.fori_loop(0, N_FIELDS, f_wd, c)

            def i_wd(i, cc):
                pltpu.make_async_copy(
                    n_v.at[i], out_ref.at[pl.ds(base, chunk), N_FIELDS + i],
                    wsem).wait()
                return cc

            lax.fori_loop(0, N_NUM, i_wd, c)

        def chunk_body(ci, carry):
            base = wid * rows_per_w + ci * chunk

            pltpu.sync_copy(xcat_ref.at[pl.ds(base * N_FIELDS,
                                              chunk * N_FIELDS)], ridx_v)
            pltpu.sync_copy(xnum_ref.at[pl.ds(base * N_NUM,
                                              chunk * N_NUM)], x_v)

            def f_off(f, c):
                def k_body(k, cc):
                    pos = iota26 + (k * LANES * N_FIELDS + f)
                    vals = plsc.load_gather(ridx_v, [pos])
                    fidx_v[f, pl.ds(k * LANES, LANES)] = vals + f * VOCAB
                    return cc

                return lax.fori_loop(0, chunk // LANES, k_body, c)

            lax.fori_loop(0, N_FIELDS, f_off, carry)

            @pl.when(ci > 0)
            def _():
                drain_writes(base - chunk, carry)

            def f_fire(f, c):
                pltpu.async_copy(tbl_ref.at[fidx_v.at[f]], g_v.at[f], gsem)
                return c

            lax.fori_loop(0, N_FIELDS, f_fire, carry)

            def i_body(i, c):
                def r_body(r, cc):
                    pp = jnp.full((LANES,), r * N_NUM + i, jnp.int32)
                    s = plsc.load_gather(x_v, [pp])
                    n_v[i, r, pl.ds(0, LANES)] = s * wlo + blo
                    n_v[i, r, pl.ds(LANES, LANES)] = s * whi + bhi
                    return cc

                return lax.fori_loop(0, chunk, r_body, c)

            lax.fori_loop(0, N_NUM, i_body, carry)

            def i_out(i, c):
                pltpu.async_copy(n_v.at[i],
                                 out_ref.at[pl.ds(base, chunk), N_FIELDS + i],
                                 wsem)
                return c

            lax.fori_loop(0, N_NUM, i_out, carry)

            def f_drain(f, c):
                pltpu.make_async_copy(
                    tbl_ref.at[fidx_v.at[f]], g_v.at[f], gsem).wait()
                return c

            lax.fori_loop(0, N_FIELDS, f_drain, carry)

            def f_out(f, c):
                pltpu.async_copy(g_v.at[f],
                                 out_ref.at[pl.ds(base, chunk), f], wsem)
                return c

            lax.fori_loop(0, N_FIELDS, f_out, carry)
            return carry

        lax.fori_loop(0, n_chunks, chunk_body, 0)
        drain_writes((wid + 1) * rows_per_w - chunk, 0)

    call = pl.kernel(
        body,
        out_type=jax.ShapeDtypeStruct((batch, N_TOK, EMBED), jnp.float32),
        mesh=mesh,
        scratch_types=[
            pltpu.VMEM((chunk * N_FIELDS,), jnp.int32),
            pltpu.VMEM((N_FIELDS, chunk), jnp.int32),
            pltpu.VMEM((chunk * N_NUM,), jnp.float32),
            pltpu.VMEM((N_FIELDS, chunk, EMBED), jnp.float32),
            pltpu.VMEM((N_NUM, chunk, EMBED), jnp.float32),
            pltpu.VMEM((EMBED,), jnp.float32),
            pltpu.VMEM((EMBED,), jnp.float32),
            pltpu.SemaphoreType.DMA,
            pltpu.SemaphoreType.DMA,
        ],
        compiler_params=pltpu.CompilerParams(
            use_tc_tiling_on_sc=False, needs_layout_passes=False),
    )
    return call(xcat_flat, xnum_flat, tbl, w, bvec)


def kernel(x_cat, x_num, tables, W, b):
    batch = x_cat.shape[0]
    xcat_flat = x_cat.astype(jnp.int32).reshape(-1)
    xnum_flat = x_num.reshape(-1)
    tbl = tables.reshape(N_FIELDS * VOCAB, EMBED)
    w = W.reshape(EMBED)
    return _run(xcat_flat, xnum_flat, tbl, w, b, batch=batch)


# speedup vs baseline: 1.4590x; 1.4590x over previous
import functools

import jax
import jax.numpy as jnp
from jax import lax
from jax.experimental import pallas as pl
from jax.experimental.pallas import tpu as pltpu
from jax.experimental.pallas import tpu_sc as plsc

N_FIELDS = 26
VOCAB = 100000
EMBED = 32
N_NUM = 13
N_TOK = N_FIELDS + N_NUM
LANES = 16


@functools.partial(jax.jit, static_argnames=("batch",))
def _run(xcat_t, xnum_t, tbl_t, w, bvec, *, batch):
    info = plsc.get_sparse_core_info()
    nc, ns = info.num_cores, info.num_subcores
    nw = nc * ns
    assert nw == EMBED, "one worker per embed lane"
    qb = batch // 4
    nblk = qb // LANES

    mesh = plsc.VectorSubcoreMesh(core_axis_name="c", subcore_axis_name="s")

    def body(xcat_ref, xnum_ref, tbl_ref, w_ref, b_ref, out_ref,
             plane_v, idx_v, oq_v, w_v, b_v, wsem):
        e = lax.axis_index("s") * nc + lax.axis_index("c")

        pltpu.sync_copy(w_ref, w_v)
        pltpu.sync_copy(b_ref, b_v)
        ee = jnp.full((LANES,), e, jnp.int32)
        we = plsc.load_gather(w_v, [ee])
        be = plsc.load_gather(b_v, [ee])

        def plane_body(t, carry):
            is_cat = t < N_FIELDS

            @pl.when(is_cat)
            def _():
                pltpu.sync_copy(xcat_ref.at[t], idx_v)
                pltpu.sync_copy(tbl_ref.at[t, e], plane_v)

            @pl.when(jnp.logical_not(is_cat))
            def _():
                pltpu.sync_copy(xnum_ref.at[t - N_FIELDS],
                                plane_v.at[pl.ds(0, batch)])

            def q_body(q, c):
                qbuf = oq_v.at[q % 2]

                @pl.when(t * 4 + q >= 2)
                def _():
                    pltpu.make_async_copy(
                        qbuf, out_ref.at[t, e, pl.ds(0, qb)], wsem).wait()

                @pl.when(is_cat)
                def _():
                    def blk(k, cc):
                        sl = pl.ds(q * qb + k * LANES, LANES)
                        idx16 = idx_v[sl]
                        s = plsc.load_gather(plane_v, [idx16])
                        qbuf[pl.ds(k * LANES, LANES)] = s
                        return cc

                    lax.fori_loop(0, nblk, blk, 0)

                @pl.when(jnp.logical_not(is_cat))
                def _():
                    def blk(k, cc):
                        sl = pl.ds(q * qb + k * LANES, LANES)
                        v16 = plane_v[sl]
                        qbuf[pl.ds(k * LANES, LANES)] = v16 * we + be
                        return cc

                    lax.fori_loop(0, nblk, blk, 0)

                pltpu.async_copy(qbuf, out_ref.at[t, e, pl.ds(q * qb, qb)],
                                 wsem)
                return c

            lax.fori_loop(0, 4, q_body, carry)
            return carry

        lax.fori_loop(0, N_TOK, plane_body, 0)

        def final_drain(q, c):
            pltpu.make_async_copy(
                oq_v.at[q], out_ref.at[N_TOK - 1, e, pl.ds(0, qb)],
                wsem).wait()
            return c

        lax.fori_loop(0, 2, final_drain, 0)

    call = pl.kernel(
        body,
        out_type=jax.ShapeDtypeStruct((N_TOK, EMBED, batch), jnp.float32),
        mesh=mesh,
        scratch_types=[
            pltpu.VMEM((VOCAB,), jnp.float32),
            pltpu.VMEM((batch,), jnp.int32),
            pltpu.VMEM((2, qb), jnp.float32),
            pltpu.VMEM((EMBED,), jnp.float32),
            pltpu.VMEM((EMBED,), jnp.float32),
            pltpu.SemaphoreType.DMA,
        ],
        compiler_params=pltpu.CompilerParams(
            use_tc_tiling_on_sc=False, needs_layout_passes=False),
    )
    return call(xcat_t, xnum_t, tbl_t, w, bvec)


def kernel(x_cat, x_num, tables, W, b):
    batch = x_cat.shape[0]
    xcat_t = x_cat.astype(jnp.int32).T
    xnum_t = x_num.T
    tbl_t = jnp.transpose(tables, (0, 2, 1))
    w = W.reshape(EMBED)
    out_t = _run(xcat_t, xnum_t, tbl_t, w, b, batch=batch)
    return jnp.transpose(out_t, (2, 0, 1))


# speedup vs baseline: 1.4846x; 1.0175x over previous
import functools

import jax
import jax.numpy as jnp
from jax import lax
from jax.experimental import pallas as pl
from jax.experimental.pallas import tpu as pltpu
from jax.experimental.pallas import tpu_sc as plsc

N_FIELDS = 26
VOCAB = 100000
EMBED = 32
N_NUM = 13
N_TOK = N_FIELDS + N_NUM
LANES = 16


@functools.partial(jax.jit, static_argnames=("batch",))
def _run(xcat_t, xnum_t, tbl_t, w, bvec, *, batch):
    info = plsc.get_sparse_core_info()
    nc, ns = info.num_cores, info.num_subcores
    nw = nc * ns
    assert nw == EMBED, "one worker per embed lane"
    qb = batch // 4
    nblk = qb // LANES
    UNROLL = 8

    mesh = plsc.VectorSubcoreMesh(core_axis_name="c", subcore_axis_name="s")

    def body(xcat_ref, xnum_ref, tbl_ref, w_ref, b_ref, out_ref,
             plane_v, idx_v, oq_v, w_v, b_v, wsem):
        e = lax.axis_index("s") * nc + lax.axis_index("c")

        pltpu.sync_copy(w_ref, w_v)
        pltpu.sync_copy(b_ref, b_v)
        ee = jnp.full((LANES,), e, jnp.int32)
        we = plsc.load_gather(w_v, [ee])
        be = plsc.load_gather(b_v, [ee])

        def plane_body(t, carry):
            is_cat = t < N_FIELDS

            @pl.when(is_cat)
            def _():
                pltpu.sync_copy(xcat_ref.at[t], idx_v)
                pltpu.sync_copy(tbl_ref.at[t, e], plane_v)

            @pl.when(jnp.logical_not(is_cat))
            def _():
                pltpu.sync_copy(xnum_ref.at[t - N_FIELDS],
                                plane_v.at[pl.ds(0, batch)])

            def q_body(q, c):
                qbuf = oq_v.at[q % 2]

                @pl.when(t * 4 + q >= 2)
                def _():
                    pltpu.make_async_copy(
                        qbuf, out_ref.at[t, e, pl.ds(0, qb)], wsem).wait()

                @pl.when(is_cat)
                def _():
                    def blk(k, cc):
                        for u in range(UNROLL):
                            o = k * UNROLL + u
                            sl = pl.ds(q * qb + o * LANES, LANES)
                            idx16 = idx_v[sl]
                            s = plsc.load_gather(plane_v, [idx16])
                            qbuf[pl.ds(o * LANES, LANES)] = s
                        return cc

                    lax.fori_loop(0, nblk // UNROLL, blk, 0)

                @pl.when(jnp.logical_not(is_cat))
                def _():
                    def blk(k, cc):
                        for u in range(UNROLL):
                            o = k * UNROLL + u
                            sl = pl.ds(q * qb + o * LANES, LANES)
                            v16 = plane_v[sl]
                            qbuf[pl.ds(o * LANES, LANES)] = v16 * we + be
                        return cc

                    lax.fori_loop(0, nblk // UNROLL, blk, 0)

                pltpu.async_copy(qbuf, out_ref.at[t, e, pl.ds(q * qb, qb)],
                                 wsem)
                return c

            lax.fori_loop(0, 4, q_body, carry)
            return carry

        lax.fori_loop(0, N_TOK, plane_body, 0)

        def final_drain(q, c):
            pltpu.make_async_copy(
                oq_v.at[q], out_ref.at[N_TOK - 1, e, pl.ds(0, qb)],
                wsem).wait()
            return c

        lax.fori_loop(0, 2, final_drain, 0)

    call = pl.kernel(
        body,
        out_type=jax.ShapeDtypeStruct((N_TOK, EMBED, batch), jnp.float32),
        mesh=mesh,
        scratch_types=[
            pltpu.VMEM((VOCAB,), jnp.float32),
            pltpu.VMEM((batch,), jnp.int32),
            pltpu.VMEM((2, qb), jnp.float32),
            pltpu.VMEM((EMBED,), jnp.float32),
            pltpu.VMEM((EMBED,), jnp.float32),
            pltpu.SemaphoreType.DMA,
        ],
        compiler_params=pltpu.CompilerParams(
            use_tc_tiling_on_sc=False, needs_layout_passes=False),
    )
    return call(xcat_t, xnum_t, tbl_t, w, bvec)


def kernel(x_cat, x_num, tables, W, b):
    batch = x_cat.shape[0]
    xcat_t = x_cat.astype(jnp.int32).T
    xnum_t = x_num.T
    tbl_t = jnp.transpose(tables, (0, 2, 1))
    w = W.reshape(EMBED)
    out_t = _run(xcat_t, xnum_t, tbl_t, w, b, batch=batch)
    return jnp.transpose(out_t, (2, 0, 1))
